# Initial kernel scaffold; baseline (speedup 1.0000x reference)
import jax
import jax.numpy as jnp
from jax.experimental import pallas as pl


def kernel(x, codebooks):
    raise NotImplementedError("write your pallas kernel here")


# speedup vs baseline: 1.1652x; 1.1652x over previous
import jax
import jax.numpy as jnp
from jax.experimental import pallas as pl
from jax.experimental.pallas import tpu as pltpu

GROUPS = 2
NQ = 4
K = 1024
D = 256
B, T = 16, 576
ROWS = B * T
R = 1152
NT = ROWS // R


def _vq_body(x_ref, cb_ref, c2_ref, q_ref, idx_ref, loss_ref):
    i = pl.program_id(1)

    @pl.when(i == 0)
    def _init():
        loss_ref[...] = jnp.zeros_like(loss_ref)

    x0 = x_ref[...]
    r = x0
    iota = jax.lax.broadcasted_iota(jnp.int32, (R, K), 1)
    qrow = jax.lax.broadcasted_iota(jnp.int32, (8, 128), 0)
    loss_acc = jnp.zeros((8, 128), jnp.float32)
    for q in range(NQ):
        cbq = cb_ref[0, q]
        c2row = c2_ref[0, q:q + 1, :]
        rowsq = jnp.sum(r * r, axis=1, keepdims=True)
        fc = jnp.dot(r, cbq.T, preferred_element_type=jnp.float32)
        dist = (rowsq - 2.0 * fc) + c2row
        min_d = jnp.min(dist, axis=1, keepdims=True)
        cand = jnp.where(dist == min_d, iota, K)
        idx = jnp.min(cand, axis=1, keepdims=True)
        idx_ref[0, :, q:q + 1] = idx
        onehot = (iota == idx).astype(jnp.float32)
        z = jnp.dot(onehot, cbq, precision=jax.lax.Precision.HIGHEST,
                    preferred_element_type=jnp.float32)
        r = r - z
        lq = jnp.sum(r * r)
        loss_acc = loss_acc + jnp.where(qrow == q, lq, 0.0)
    q_ref[...] = x0 - r
    loss_ref[0] += loss_acc


def kernel(x, codebooks):
    xf = x.reshape(ROWS, GROUPS * D)
    c2 = jnp.sum(codebooks * codebooks, axis=-1)
    c2p = jnp.concatenate(
        [c2, jnp.zeros((GROUPS, 8 - NQ, K), jnp.float32)], axis=1)

    grid = (GROUPS, NT)
    qflat, idx_out, loss_out = pl.pallas_call(
        _vq_body,
        grid=grid,
        in_specs=[
            pl.BlockSpec((R, D), lambda g, i: (i, g)),
            pl.BlockSpec((1, NQ, K, D), lambda g, i: (g, 0, 0, 0)),
            pl.BlockSpec((1, 8, K), lambda g, i: (g, 0, 0)),
        ],
        out_specs=[
            pl.BlockSpec((R, D), lambda g, i: (i, g)),
            pl.BlockSpec((1, R, 8), lambda g, i: (g, i, 0)),
            pl.BlockSpec((1, 8, 128), lambda g, i: (g, 0, 0)),
        ],
        out_shape=[
            jax.ShapeDtypeStruct((ROWS, GROUPS * D), jnp.float32),
            jax.ShapeDtypeStruct((GROUPS, ROWS, 8), jnp.int32),
            jax.ShapeDtypeStruct((GROUPS, 8, 128), jnp.float32),
        ],
    )(xf, codebooks, c2p)

    quantized = qflat.reshape(B, T, GROUPS * D)
    all_indices = jnp.transpose(idx_out, (0, 2, 1))[:, :NQ].reshape(
        GROUPS, NQ, B, T)
    commit_losses = 1.25 * jnp.sum(loss_out[:, :NQ, 0], axis=1) / (ROWS * D)
    return quantized, all_indices, commit_losses
